# baseline (speedup 1.0000x reference)
import jax
import jax.numpy as jnp
from jax import lax
from jax.experimental import pallas as pl
from jax.experimental.pallas import tpu as pltpu
from jax.experimental.pallas import tpu_sc as plsc
import functools

B = 4096
H = 50
D = 64
L = 16
NW = 32
RPW = B // NW
NBUF = 2


def _body(feat_hbm, w_hbm, table_hbm, out_hbm, idx_v, wv, buf0, buf1, out_v,
          sem0, sem1):
    wid = lax.axis_index("s") * 2 + lax.axis_index("c")
    base = wid * RPW

    pltpu.sync_copy(feat_hbm.at[pl.ds(base, RPW)], idx_v)
    pltpu.sync_copy(w_hbm.at[pl.ds(base * H, RPW * H)], wv)

    bufs = (buf0, buf1)
    sems = (sem0, sem1)

    for k in range(NBUF):
        pltpu.async_copy(table_hbm.at[idx_v.at[k]], bufs[k], sems[k])

    def step(i, carry):
        b0 = i * NBUF
        for k in range(NBUF):
            b = b0 + k
            buf, sem = bufs[k], sems[k]
            pltpu.make_async_copy(table_hbm.at[idx_v.at[b]], buf, sem).wait()
            wbase = b * H
            wregs = [wv[pl.ds(wbase + o, L)] for o in (0, 16, 32, 34)]
            acc = [jnp.zeros((L,), jnp.float32) for _ in range(D // L)]
            for j in range(H):
                reg, lane = (wregs[j // 16], j % 16) if j < 48 else (wregs[3], j - 34)
                w = reg.at[jnp.full((L,), lane, jnp.int32)].get(
                    mode="promise_in_bounds")
                for d in range(D // L):
                    acc[d] = acc[d] + buf[j, pl.ds(L * d, L)] * w
            for d in range(D // L):
                out_v[b, pl.ds(L * d, L)] = acc[d]
            nb = b + NBUF

            @pl.when(nb < RPW)
            def _():
                pltpu.async_copy(table_hbm.at[idx_v.at[nb]], buf, sem)
        return carry

    lax.fori_loop(0, RPW // NBUF, step, 0)

    pltpu.sync_copy(out_v, out_hbm.at[pl.ds(base, RPW)])


@jax.jit
def kernel(features, weights, table):
    mesh = plsc.VectorSubcoreMesh(core_axis_name="c", subcore_axis_name="s")
    run = pl.kernel(
        _body,
        out_type=jax.ShapeDtypeStruct((B, D), jnp.float32),
        mesh=mesh,
        scratch_types=[
            pltpu.VMEM((RPW, H), jnp.int32),
            pltpu.VMEM((RPW * H,), jnp.float32),
            pltpu.VMEM((H, D), jnp.float32),
            pltpu.VMEM((H, D), jnp.float32),
            pltpu.VMEM((RPW, D), jnp.float32),
            pltpu.SemaphoreType.DMA,
            pltpu.SemaphoreType.DMA,
        ],
        compiler_params=pltpu.CompilerParams(use_tc_tiling_on_sc=False),
    )
    return run(features, weights.reshape(B * H), table)


# speedup vs baseline: 1.0009x; 1.0009x over previous
import jax
import jax.numpy as jnp
from jax import lax
from jax.experimental import pallas as pl
from jax.experimental.pallas import tpu as pltpu
from jax.experimental.pallas import tpu_sc as plsc
import functools

B = 4096
H = 50
D = 64
L = 16
NW = 32
RPW = B // NW
NBUF = 2


def _body(feat_hbm, w_hbm, table_hbm, out_hbm, idx_v, wv, buf0, buf1, out_v,
          sem0, sem1):
    wid = lax.axis_index("s") * 2 + lax.axis_index("c")
    base = wid * RPW

    pltpu.sync_copy(feat_hbm.at[pl.ds(base, RPW)], idx_v)
    pltpu.sync_copy(w_hbm.at[pl.ds(base, RPW)], wv)

    bufs = (buf0, buf1)
    sems = (sem0, sem1)

    for k in range(NBUF):
        pltpu.async_copy(table_hbm.at[idx_v.at[k]], bufs[k], sems[k])

    def step(i, carry):
        b0 = i * NBUF
        for k in range(NBUF):
            b = b0 + k
            buf, sem = bufs[k], sems[k]
            pltpu.make_async_copy(table_hbm.at[idx_v.at[b]], buf, sem).wait()
            wregs = [wv[b, pl.ds(o, L)] for o in (0, 16, 32, 34)]
            acc = [jnp.zeros((L,), jnp.float32) for _ in range(D // L)]
            for j in range(H):
                reg, lane = (wregs[j // 16], j % 16) if j < 48 else (wregs[3], j - 34)
                w = reg.at[jnp.full((L,), lane, jnp.int32)].get(
                    mode="promise_in_bounds")
                for d in range(D // L):
                    acc[d] = acc[d] + buf[j, pl.ds(L * d, L)] * w
            for d in range(D // L):
                out_v[b, pl.ds(L * d, L)] = acc[d]
            nb = b + NBUF

            @pl.when(nb < RPW)
            def _():
                pltpu.async_copy(table_hbm.at[idx_v.at[nb]], buf, sem)
        return carry

    lax.fori_loop(0, RPW // NBUF, step, 0)

    pltpu.sync_copy(out_v, out_hbm.at[pl.ds(base, RPW)])


@jax.jit
def kernel(features, weights, table):
    mesh = plsc.VectorSubcoreMesh(core_axis_name="c", subcore_axis_name="s")
    run = pl.kernel(
        _body,
        out_type=jax.ShapeDtypeStruct((B, D), jnp.float32),
        mesh=mesh,
        scratch_types=[
            pltpu.VMEM((RPW, H), jnp.int32),
            pltpu.VMEM((RPW, H), jnp.float32),
            pltpu.VMEM((H, D), jnp.float32),
            pltpu.VMEM((H, D), jnp.float32),
            pltpu.VMEM((RPW, D), jnp.float32),
            pltpu.SemaphoreType.DMA,
            pltpu.SemaphoreType.DMA,
        ],
        compiler_params=pltpu.CompilerParams(use_tc_tiling_on_sc=False),
    )
    return run(features, weights, table)
